# baseline (speedup 1.0000x reference)
import functools

import jax
import jax.numpy as jnp
from jax import lax
from jax.experimental import pallas as pl
from jax.experimental.pallas import tpu as pltpu
from jax.experimental.pallas import tpu_sc as plsc

N_NODES = 10000
NUM_CORES = 2
NUM_SUBCORES = 16
NUM_WORKERS = NUM_CORES * NUM_SUBCORES
CHUNK = 112
N_PAD = 10240
ROWS_PER_SUB = N_PAD // NUM_SUBCORES


NBUF = 3
NSRC = 3
NDST = 4
NWGT = 3
UNROLL = 12


@functools.lru_cache(maxsize=None)
def _make_spmm(feat: int, n_chunks_w: int):
    assert n_chunks_w % UNROLL == 0
    n_outer = n_chunks_w // UNROLL
    mesh = plsc.VectorSubcoreMesh(core_axis_name="c", subcore_axis_name="s")

    @functools.partial(
        pl.kernel,
        out_type=jax.ShapeDtypeStruct((NUM_CORES, N_PAD, feat), jnp.float32),
        mesh=mesh,
        scratch_types=[
            pltpu.VMEM((NSRC, CHUNK), jnp.int32),
            pltpu.VMEM((NDST, CHUNK), jnp.int32),
            pltpu.VMEM((NWGT, CHUNK), jnp.float32),
            pltpu.VMEM((NBUF, CHUNK, feat), jnp.float32),
            pltpu.VMEM_SHARED((N_PAD, feat), jnp.float32),
        ] + [pltpu.SemaphoreType.DMA] * (NBUF + NBUF + NSRC + NDST + NWGT),
    )
    def spmm(sup_hbm, src_hbm, dst_hbm, w_hbm, zeros_hbm, out_hbm,
             src_v, dst_v, w_v, rows_v, acc, *sems):
        gsem = sems[:NBUF]
        ssem = sems[NBUF:2 * NBUF]
        isrc = sems[2 * NBUF:2 * NBUF + NSRC]
        idst = sems[2 * NBUF + NSRC:2 * NBUF + NSRC + NDST]
        iwgt = sems[2 * NBUF + NSRC + NDST:]
        c = lax.axis_index("c")
        s = lax.axis_index("s")
        wid = c * NUM_SUBCORES + s
        base_e = wid * n_chunks_w * CHUNK

        def stage(i, ps, pd, pw):
            off = base_e + i * CHUNK
            pltpu.async_copy(src_hbm.at[pl.ds(off, CHUNK)], src_v.at[ps],
                             isrc[ps])
            pltpu.async_copy(dst_hbm.at[pl.ds(off, CHUNK)], dst_v.at[pd],
                             idst[pd])
            pltpu.async_copy(w_hbm.at[pl.ds(off, CHUNK)], w_v.at[pw],
                             iwgt[pw])

        def src_wait(ps):
            pltpu.make_async_copy(src_hbm.at[pl.ds(0, CHUNK)], src_v.at[ps],
                                  isrc[ps]).wait()

        def dst_wait(pd):
            pltpu.make_async_copy(dst_hbm.at[pl.ds(0, CHUNK)], dst_v.at[pd],
                                  idst[pd]).wait()

        def wgt_wait(pw):
            pltpu.make_async_copy(w_hbm.at[pl.ds(0, CHUNK)], w_v.at[pw],
                                  iwgt[pw]).wait()

        def gather_start(b, ps):
            pltpu.async_copy(sup_hbm.at[src_v.at[ps]], rows_v.at[b], gsem[b])

        def gather_wait(b, ps):
            pltpu.make_async_copy(sup_hbm.at[src_v.at[ps]], rows_v.at[b],
                                  gsem[b]).wait()

        def scatter_start(b, pd):
            pltpu.async_copy(rows_v.at[b], acc.at[dst_v.at[pd]], ssem[b],
                             add=True)

        def scatter_wait(b, pd):
            pltpu.make_async_copy(rows_v.at[b], acc.at[dst_v.at[pd]],
                                  ssem[b]).wait()

        def scale(b, pw):
            def group_body(g, carry):
                w16 = w_v[pw, pl.ds(g * 16, 16)]
                for j in range(16):
                    we = w16[j]
                    e = g * 16 + j
                    for f in range(feat // 16):
                        sl = pl.ds(f * 16, 16)
                        rows_v[b, e, sl] = rows_v[b, e, sl] * we
                return carry

            lax.fori_loop(0, CHUNK // 16, group_body, 0)

        pltpu.sync_copy(zeros_hbm, acc.at[pl.ds(s * ROWS_PER_SUB, ROWS_PER_SUB)])
        plsc.subcore_barrier()

        stage(0, 0, 0, 0)
        stage(1, 1, 1, 1)
        src_wait(0)
        gather_start(0, 0)

        def outer(t, carry):
            for p in range(UNROLL):
                i = t * UNROLL + p
                b = p % NBUF
                bn = (p + 1) % NBUF
                if p >= 2:
                    scatter_wait(bn, (p - 2) % NDST)
                else:
                    @pl.when(t >= 1)
                    def _():
                        scatter_wait(bn, (p - 2) % NDST)
                if p < UNROLL - 2:
                    stage(i + 2, (p + 2) % NSRC, (p + 2) % NDST,
                          (p + 2) % NWGT)
                else:
                    @pl.when(t < n_outer - 1)
                    def _():
                        stage(i + 2, (p + 2) % NSRC, (p + 2) % NDST,
                              (p + 2) % NWGT)
                if p < UNROLL - 1:
                    src_wait((p + 1) % NSRC)
                    gather_start(bn, (p + 1) % NSRC)
                else:
                    @pl.when(t < n_outer - 1)
                    def _():
                        src_wait((p + 1) % NSRC)
                        gather_start(bn, (p + 1) % NSRC)
                gather_wait(b, p % NSRC)
                wgt_wait(p % NWGT)
                scale(b, p % NWGT)
                dst_wait(p % NDST)
                scatter_start(b, p % NDST)
            return carry

        lax.fori_loop(0, n_outer, outer, 0)

        scatter_wait((n_chunks_w - 2) % NBUF, (n_chunks_w - 2) % NDST)
        scatter_wait((n_chunks_w - 1) % NBUF, (n_chunks_w - 1) % NDST)

        plsc.subcore_barrier()

        pltpu.sync_copy(acc.at[pl.ds(s * ROWS_PER_SUB, ROWS_PER_SUB)],
                        out_hbm.at[c, pl.ds(s * ROWS_PER_SUB, ROWS_PER_SUB)])

    return spmm


def _blk(n):
    return 1024 if n % 1024 == 0 else 1000


def _tc_matmul(x, w):
    n, k = x.shape
    m = w.shape[1]
    _BLK = _blk(n)

    def body(x_ref, w_ref, o_ref):
        o_ref[...] = jnp.dot(x_ref[...], w_ref[...],
                             preferred_element_type=jnp.float32)

    return pl.pallas_call(
        body,
        grid=(n // _BLK,),
        in_specs=[pl.BlockSpec((_BLK, k), lambda i: (i, 0)),
                  pl.BlockSpec((k, m), lambda i: (0, 0))],
        out_specs=pl.BlockSpec((_BLK, m), lambda i: (i, 0)),
        out_shape=jax.ShapeDtypeStruct((n, m), jnp.float32),
    )(x, w)


def _tc_add_relu_matmul(p0, p1, b, w):
    n, k = p0.shape
    m = w.shape[1]
    _BLK = _blk(n)

    def body(p0_ref, p1_ref, b_ref, w_ref, o_ref):
        h = jnp.maximum(p0_ref[...] + p1_ref[...] + b_ref[...], 0.0)
        o_ref[...] = jnp.dot(h, w_ref[...], preferred_element_type=jnp.float32)

    return pl.pallas_call(
        body,
        grid=(n // _BLK,),
        in_specs=[pl.BlockSpec((_BLK, k), lambda i: (i, 0)),
                  pl.BlockSpec((_BLK, k), lambda i: (i, 0)),
                  pl.BlockSpec((1, k), lambda i: (0, 0)),
                  pl.BlockSpec((k, m), lambda i: (0, 0))],
        out_specs=pl.BlockSpec((_BLK, m), lambda i: (i, 0)),
        out_shape=jax.ShapeDtypeStruct((n, m), jnp.float32),
    )(p0, p1, b, w)


def _tc_bias_softmax(p0, p1, b, ncls):
    n, k = p0.shape
    _BLK = _blk(n)

    def body(p0_ref, p1_ref, b_ref, o_ref):
        z = p0_ref[:, :ncls] + p1_ref[:, :ncls] + b_ref[...]
        z = z - jnp.max(z, axis=1, keepdims=True)
        e = jnp.exp(z)
        o_ref[...] = e / jnp.sum(e, axis=1, keepdims=True)

    return pl.pallas_call(
        body,
        grid=(n // _BLK,),
        in_specs=[pl.BlockSpec((_BLK, k), lambda i: (i, 0)),
                  pl.BlockSpec((_BLK, k), lambda i: (i, 0)),
                  pl.BlockSpec((1, ncls), lambda i: (0, 0))],
        out_specs=pl.BlockSpec((_BLK, ncls), lambda i: (i, 0)),
        out_shape=jax.ShapeDtypeStruct((n, ncls), jnp.float32),
    )(p0, p1, b)


def kernel(x, edge_index, edge_weight, W1, b1, W2, b2):
    src = edge_index[0].astype(jnp.int32)
    dst = edge_index[1].astype(jnp.int32)
    ew = edge_weight.astype(jnp.float32)

    e = src.shape[0]
    tile = NUM_WORKERS * CHUNK * UNROLL
    e_pad = ((e + tile - 1) // tile) * tile
    pad = e_pad - e
    src_p = jnp.pad(src, (0, pad))
    dst_p = jnp.pad(dst, (0, pad))
    ew_p = jnp.pad(ew, (0, pad))
    n_chunks_w = e_pad // (NUM_WORKERS * CHUNK)

    nhid = W1.shape[1]
    ncls = W2.shape[1]
    zeros_h = jnp.zeros((ROWS_PER_SUB, nhid), jnp.float32)
    spmm = _make_spmm(nhid, n_chunks_w)
    w2_pad = jnp.pad(W2, ((0, 0), (0, nhid - ncls)))

    support = _tc_matmul(x, W1)
    parts1 = spmm(support, src_p, dst_p, ew_p, zeros_h)
    support2 = _tc_add_relu_matmul(parts1[0], parts1[1], b1.reshape(1, -1),
                                   w2_pad)
    parts2 = spmm(support2, src_p, dst_p, ew_p, zeros_h)
    out = _tc_bias_softmax(parts2[0], parts2[1], b2.reshape(1, -1), ncls)
    return out[:N_NODES]


# speedup vs baseline: 7.9084x; 7.9084x over previous
import functools

import jax
import jax.numpy as jnp
from jax import lax
from jax.experimental import pallas as pl
from jax.experimental.pallas import tpu as pltpu
from jax.experimental.pallas import tpu_sc as plsc

N_NODES = 10000
NUM_CORES = 2
NUM_SUBCORES = 16
NUM_WORKERS = NUM_CORES * NUM_SUBCORES
CHUNK = 112
N_PAD = 10240
ROWS_PER_SUB = N_PAD // NUM_SUBCORES


NBUF = 3
NSRC = 3
NDST = 4
NWGT = 3
UNROLL = 12


@functools.lru_cache(maxsize=None)
def _make_spmm(feat: int, n_chunks_w: int):
    assert n_chunks_w % UNROLL == 0
    n_outer = n_chunks_w // UNROLL
    mesh = plsc.VectorSubcoreMesh(core_axis_name="c", subcore_axis_name="s")

    @functools.partial(
        pl.kernel,
        out_type=jax.ShapeDtypeStruct((NUM_CORES, N_PAD, feat), jnp.float32),
        mesh=mesh,
        scratch_types=[
            pltpu.VMEM((NSRC, CHUNK), jnp.int32),
            pltpu.VMEM((NDST, CHUNK), jnp.int32),
            pltpu.VMEM((NWGT, CHUNK), jnp.float32),
            pltpu.VMEM((NBUF, CHUNK, feat), jnp.float32),
            pltpu.VMEM_SHARED((N_PAD, feat), jnp.float32),
        ] + [pltpu.SemaphoreType.DMA] * (NBUF + NBUF + NSRC + NDST + NWGT),
    )
    def spmm(sup_hbm, src_hbm, dst_hbm, w_hbm, zeros_hbm, out_hbm,
             src_v, dst_v, w_v, rows_v, acc, *sems):
        gsem = sems[:NBUF]
        ssem = sems[NBUF:2 * NBUF]
        isrc = sems[2 * NBUF:2 * NBUF + NSRC]
        idst = sems[2 * NBUF + NSRC:2 * NBUF + NSRC + NDST]
        iwgt = sems[2 * NBUF + NSRC + NDST:]
        c = lax.axis_index("c")
        s = lax.axis_index("s")
        wid = c * NUM_SUBCORES + s
        base_e = wid * n_chunks_w * CHUNK

        def stage(i, ps, pd, pw):
            off = base_e + i * CHUNK
            pltpu.async_copy(src_hbm.at[pl.ds(off, CHUNK)], src_v.at[ps],
                             isrc[ps])
            pltpu.async_copy(dst_hbm.at[pl.ds(off, CHUNK)], dst_v.at[pd],
                             idst[pd])
            pltpu.async_copy(w_hbm.at[pl.ds(off, CHUNK)], w_v.at[pw],
                             iwgt[pw])

        def src_wait(ps):
            pltpu.make_async_copy(src_hbm.at[pl.ds(0, CHUNK)], src_v.at[ps],
                                  isrc[ps]).wait()

        def dst_wait(pd):
            pltpu.make_async_copy(dst_hbm.at[pl.ds(0, CHUNK)], dst_v.at[pd],
                                  idst[pd]).wait()

        def wgt_wait(pw):
            pltpu.make_async_copy(w_hbm.at[pl.ds(0, CHUNK)], w_v.at[pw],
                                  iwgt[pw]).wait()

        def gather_start(b, ps):
            pltpu.async_copy(sup_hbm.at[src_v.at[ps]], rows_v.at[b], gsem[b])

        def gather_wait(b, ps):
            pltpu.make_async_copy(sup_hbm.at[src_v.at[ps]], rows_v.at[b],
                                  gsem[b]).wait()

        def scatter_start(b, pd):
            pltpu.async_copy(rows_v.at[b], acc.at[dst_v.at[pd]], ssem[b],
                             add=True)

        def scatter_wait(b, pd):
            pltpu.make_async_copy(rows_v.at[b], acc.at[dst_v.at[pd]],
                                  ssem[b]).wait()

        def scale(b, pw):
            def group_body(g, carry):
                w16 = w_v[pw, pl.ds(g * 16, 16)]
                for j in range(16):
                    we = w16[j]
                    e = g * 16 + j
                    for f in range(feat // 16):
                        sl = pl.ds(f * 16, 16)
                        rows_v[b, e, sl] = rows_v[b, e, sl] * we
                return carry

            lax.fori_loop(0, CHUNK // 16, group_body, 0)

        pltpu.sync_copy(zeros_hbm, acc.at[pl.ds(s * ROWS_PER_SUB, ROWS_PER_SUB)])
        plsc.subcore_barrier()

        stage(0, 0, 0, 0)
        stage(1, 1, 1, 1)
        src_wait(0)
        gather_start(0, 0)

        def outer(t, carry):
            for p in range(UNROLL):
                i = t * UNROLL + p
                b = p % NBUF
                bn = (p + 1) % NBUF
                if p >= 2:
                    scatter_wait(bn, (p - 2) % NDST)
                else:
                    @pl.when(t >= 1)
                    def _():
                        scatter_wait(bn, (p - 2) % NDST)
                if p < UNROLL - 2:
                    stage(i + 2, (p + 2) % NSRC, (p + 2) % NDST,
                          (p + 2) % NWGT)
                else:
                    @pl.when(t < n_outer - 1)
                    def _():
                        stage(i + 2, (p + 2) % NSRC, (p + 2) % NDST,
                              (p + 2) % NWGT)
                if p < UNROLL - 1:
                    src_wait((p + 1) % NSRC)
                    gather_start(bn, (p + 1) % NSRC)
                else:
                    @pl.when(t < n_outer - 1)
                    def _():
                        src_wait((p + 1) % NSRC)
                        gather_start(bn, (p + 1) % NSRC)
                gather_wait(b, p % NSRC)
                wgt_wait(p % NWGT)
                scale(b, p % NWGT)
                dst_wait(p % NDST)
                scatter_start(b, p % NDST)
            return carry

        lax.fori_loop(0, n_outer, outer, 0)

        scatter_wait((n_chunks_w - 2) % NBUF, (n_chunks_w - 2) % NDST)
        scatter_wait((n_chunks_w - 1) % NBUF, (n_chunks_w - 1) % NDST)

        plsc.subcore_barrier()

        pltpu.sync_copy(acc.at[pl.ds(s * ROWS_PER_SUB, ROWS_PER_SUB)],
                        out_hbm.at[c, pl.ds(s * ROWS_PER_SUB, ROWS_PER_SUB)])

    return spmm


def _blk(n):
    return 1024 if n % 1024 == 0 else 1000


def _tc_matmul(x, w):
    n, k = x.shape
    m = w.shape[1]
    _BLK = _blk(n)

    def body(x_ref, w_ref, o_ref):
        o_ref[...] = jnp.dot(x_ref[...], w_ref[...],
                             preferred_element_type=jnp.float32)

    return pl.pallas_call(
        body,
        grid=(n // _BLK,),
        in_specs=[pl.BlockSpec((_BLK, k), lambda i: (i, 0)),
                  pl.BlockSpec((k, m), lambda i: (0, 0))],
        out_specs=pl.BlockSpec((_BLK, m), lambda i: (i, 0)),
        out_shape=jax.ShapeDtypeStruct((n, m), jnp.float32),
    )(x, w)


def _tc_add_relu_matmul(p0, p1, b, w):
    n, k = p0.shape
    m = w.shape[1]
    _BLK = _blk(n)

    def body(p0_ref, p1_ref, b_ref, w_ref, o_ref):
        h = jnp.maximum(p0_ref[...] + p1_ref[...] + b_ref[...], 0.0)
        o_ref[...] = jnp.dot(h, w_ref[...], preferred_element_type=jnp.float32)

    return pl.pallas_call(
        body,
        grid=(n // _BLK,),
        in_specs=[pl.BlockSpec((_BLK, k), lambda i: (i, 0)),
                  pl.BlockSpec((_BLK, k), lambda i: (i, 0)),
                  pl.BlockSpec((1, k), lambda i: (0, 0)),
                  pl.BlockSpec((k, m), lambda i: (0, 0))],
        out_specs=pl.BlockSpec((_BLK, m), lambda i: (i, 0)),
        out_shape=jax.ShapeDtypeStruct((n, m), jnp.float32),
    )(p0, p1, b, w)


def _tc_bias_softmax(p0, p1, b, ncls):
    n, k = p0.shape
    _BLK = _blk(n)

    def body(p0_ref, p1_ref, b_ref, o_ref):
        z = p0_ref[:, :ncls] + p1_ref[:, :ncls] + b_ref[...]
        z = z - jnp.max(z, axis=1, keepdims=True)
        e = jnp.exp(z)
        o_ref[...] = e / jnp.sum(e, axis=1, keepdims=True)

    return pl.pallas_call(
        body,
        grid=(n // _BLK,),
        in_specs=[pl.BlockSpec((_BLK, k), lambda i: (i, 0)),
                  pl.BlockSpec((_BLK, k), lambda i: (i, 0)),
                  pl.BlockSpec((1, ncls), lambda i: (0, 0))],
        out_specs=pl.BlockSpec((_BLK, ncls), lambda i: (i, 0)),
        out_shape=jax.ShapeDtypeStruct((n, ncls), jnp.float32),
    )(p0, p1, b)


def kernel(x, edge_index, edge_weight, W1, b1, W2, b2):
    src = edge_index[0].astype(jnp.int32)
    dst = edge_index[1].astype(jnp.int32)
    ew = edge_weight.astype(jnp.float32)

    e = src.shape[0]
    tile = NUM_WORKERS * CHUNK * UNROLL
    e_pad = ((e + tile - 1) // tile) * tile
    pad = e_pad - e
    spread = jnp.arange(pad, dtype=jnp.int32) % jnp.int32(N_NODES)
    src_p = jnp.concatenate([src, spread])
    dst_p = jnp.concatenate([dst, spread])
    ew_p = jnp.pad(ew, (0, pad))
    n_chunks_w = e_pad // (NUM_WORKERS * CHUNK)

    nhid = W1.shape[1]
    ncls = W2.shape[1]
    zeros_h = jnp.zeros((ROWS_PER_SUB, nhid), jnp.float32)
    spmm = _make_spmm(nhid, n_chunks_w)
    w2_pad = jnp.pad(W2, ((0, 0), (0, nhid - ncls)))

    support = _tc_matmul(x, W1)
    parts1 = spmm(support, src_p, dst_p, ew_p, zeros_h)
    support2 = _tc_add_relu_matmul(parts1[0], parts1[1], b1.reshape(1, -1),
                                   w2_pad)
    parts2 = spmm(support2, src_p, dst_p, ew_p, zeros_h)
    out = _tc_bias_softmax(parts2[0], parts2[1], b2.reshape(1, -1), ncls)
    return out[:N_NODES]


# speedup vs baseline: 8.3138x; 1.0513x over previous
import functools

import jax
import jax.numpy as jnp
from jax import lax
from jax.experimental import pallas as pl
from jax.experimental.pallas import tpu as pltpu
from jax.experimental.pallas import tpu_sc as plsc

N_NODES = 10000
NUM_CORES = 2
NUM_SUBCORES = 16
NUM_WORKERS = NUM_CORES * NUM_SUBCORES
CHUNK = 112
N_PAD = 10240
ROWS_PER_SUB = N_PAD // NUM_SUBCORES


NBUF = 3
NSRC = 3
NDST = 4
NWGT = 3
UNROLL = 12


@functools.lru_cache(maxsize=None)
def _make_spmm(feat: int, n_chunks_w: int):
    assert n_chunks_w % UNROLL == 0
    n_outer = n_chunks_w // UNROLL
    mesh = plsc.VectorSubcoreMesh(core_axis_name="c", subcore_axis_name="s")

    @functools.partial(
        pl.kernel,
        out_type=jax.ShapeDtypeStruct((NUM_CORES, N_PAD, feat), jnp.float32),
        mesh=mesh,
        scratch_types=[
            pltpu.VMEM((NSRC, CHUNK), jnp.int32),
            pltpu.VMEM((NDST, CHUNK), jnp.int32),
            pltpu.VMEM((NWGT, CHUNK), jnp.float32),
            pltpu.VMEM((NBUF, CHUNK, feat), jnp.float32),
            pltpu.VMEM_SHARED((N_PAD, feat), jnp.float32),
        ] + [pltpu.SemaphoreType.DMA] * (NBUF + NBUF + NSRC + NDST + NWGT),
    )
    def spmm(sup_hbm, src_hbm, dst_hbm, w_hbm, zeros_hbm, out_hbm,
             src_v, dst_v, w_v, rows_v, acc, *sems):
        gsem = sems[:NBUF]
        ssem = sems[NBUF:2 * NBUF]
        isrc = sems[2 * NBUF:2 * NBUF + NSRC]
        idst = sems[2 * NBUF + NSRC:2 * NBUF + NSRC + NDST]
        iwgt = sems[2 * NBUF + NSRC + NDST:]
        c = lax.axis_index("c")
        s = lax.axis_index("s")
        wid = c * NUM_SUBCORES + s
        base_e = wid * n_chunks_w * CHUNK

        def stage(i, ps, pd, pw):
            off = base_e + i * CHUNK
            pltpu.async_copy(src_hbm.at[pl.ds(off, CHUNK)], src_v.at[ps],
                             isrc[ps])
            pltpu.async_copy(dst_hbm.at[pl.ds(off, CHUNK)], dst_v.at[pd],
                             idst[pd])
            pltpu.async_copy(w_hbm.at[pl.ds(off, CHUNK)], w_v.at[pw],
                             iwgt[pw])

        def src_wait(ps):
            pltpu.make_async_copy(src_hbm.at[pl.ds(0, CHUNK)], src_v.at[ps],
                                  isrc[ps]).wait()

        def dst_wait(pd):
            pltpu.make_async_copy(dst_hbm.at[pl.ds(0, CHUNK)], dst_v.at[pd],
                                  idst[pd]).wait()

        def wgt_wait(pw):
            pltpu.make_async_copy(w_hbm.at[pl.ds(0, CHUNK)], w_v.at[pw],
                                  iwgt[pw]).wait()

        def gather_start(b, ps):
            pltpu.async_copy(sup_hbm.at[src_v.at[ps]], rows_v.at[b], gsem[b])

        def gather_wait(b, ps):
            pltpu.make_async_copy(sup_hbm.at[src_v.at[ps]], rows_v.at[b],
                                  gsem[b]).wait()

        def scatter_start(b, pd):
            pltpu.async_copy(rows_v.at[b], acc.at[dst_v.at[pd]], ssem[b],
                             add=True)

        def scatter_wait(b, pd):
            pltpu.make_async_copy(rows_v.at[b], acc.at[dst_v.at[pd]],
                                  ssem[b]).wait()

        def scale(b, pw):
            def group_body(g, carry):
                w16 = w_v[pw, pl.ds(g * 16, 16)]
                for j in range(16):
                    we = w16[j]
                    e = g * 16 + j
                    for f in range(feat // 16):
                        sl = pl.ds(f * 16, 16)
                        rows_v[b, e, sl] = rows_v[b, e, sl] * we
                return carry

            lax.fori_loop(0, CHUNK // 16, group_body, 0)

        pltpu.sync_copy(zeros_hbm, acc.at[pl.ds(s * ROWS_PER_SUB, ROWS_PER_SUB)])
        plsc.subcore_barrier()

        stage(0, 0, 0, 0)
        stage(1, 1, 1, 1)
        src_wait(0)
        gather_start(0, 0)

        def outer(t, carry):
            for p in range(UNROLL):
                i = t * UNROLL + p
                b = p % NBUF
                bn = (p + 1) % NBUF
                if p >= 2:
                    scatter_wait(bn, (p - 2) % NDST)
                else:
                    @pl.when(t >= 1)
                    def _():
                        scatter_wait(bn, (p - 2) % NDST)
                if p < UNROLL - 2:
                    stage(i + 2, (p + 2) % NSRC, (p + 2) % NDST,
                          (p + 2) % NWGT)
                else:
                    @pl.when(t < n_outer - 1)
                    def _():
                        stage(i + 2, (p + 2) % NSRC, (p + 2) % NDST,
                              (p + 2) % NWGT)
                if p < UNROLL - 1:
                    src_wait((p + 1) % NSRC)
                    gather_start(bn, (p + 1) % NSRC)
                else:
                    @pl.when(t < n_outer - 1)
                    def _():
                        src_wait((p + 1) % NSRC)
                        gather_start(bn, (p + 1) % NSRC)
                gather_wait(b, p % NSRC)
                wgt_wait(p % NWGT)
                scale(b, p % NWGT)
                dst_wait(p % NDST)
                scatter_start(b, p % NDST)
            return carry

        lax.fori_loop(0, n_outer, outer, 0)

        scatter_wait((n_chunks_w - 2) % NBUF, (n_chunks_w - 2) % NDST)
        scatter_wait((n_chunks_w - 1) % NBUF, (n_chunks_w - 1) % NDST)

        plsc.subcore_barrier()

        pltpu.sync_copy(acc.at[pl.ds(s * ROWS_PER_SUB, ROWS_PER_SUB)],
                        out_hbm.at[c, pl.ds(s * ROWS_PER_SUB, ROWS_PER_SUB)])

    return spmm


def _blk(n):
    return 1024 if n % 1024 == 0 else 1000


def _tc_matmul(x, w):
    n, k = x.shape
    m = w.shape[1]
    _BLK = _blk(n)

    def body(x_ref, w_ref, o_ref):
        o_ref[...] = jnp.dot(x_ref[...], w_ref[...],
                             preferred_element_type=jnp.float32)

    return pl.pallas_call(
        body,
        grid=(n // _BLK,),
        in_specs=[pl.BlockSpec((_BLK, k), lambda i: (i, 0)),
                  pl.BlockSpec((k, m), lambda i: (0, 0))],
        out_specs=pl.BlockSpec((_BLK, m), lambda i: (i, 0)),
        out_shape=jax.ShapeDtypeStruct((n, m), jnp.float32),
    )(x, w)


def _tc_add_relu_matmul(parts, b, w):
    _, n, k = parts.shape
    m = w.shape[1]
    _BLK = _blk(n)

    def body(p_ref, b_ref, w_ref, o_ref):
        p = p_ref[...]
        h = jnp.maximum(p[0] + p[1] + b_ref[...], 0.0)
        o_ref[...] = jnp.dot(h, w_ref[...], preferred_element_type=jnp.float32)

    return pl.pallas_call(
        body,
        grid=(n // _BLK,),
        in_specs=[pl.BlockSpec((2, _BLK, k), lambda i: (0, i, 0)),
                  pl.BlockSpec((1, k), lambda i: (0, 0)),
                  pl.BlockSpec((k, m), lambda i: (0, 0))],
        out_specs=pl.BlockSpec((_BLK, m), lambda i: (i, 0)),
        out_shape=jax.ShapeDtypeStruct((n, m), jnp.float32),
    )(parts, b, w)


def _tc_bias_softmax(parts, b, n_out, ncls):
    _, n, k = parts.shape
    _BLK = _blk(n)

    def body(p_ref, b_ref, o_ref):
        p = p_ref[...]
        z = p[0, :, :ncls] + p[1, :, :ncls] + b_ref[...]
        z = z - jnp.max(z, axis=1, keepdims=True)
        e = jnp.exp(z)
        o_ref[...] = e / jnp.sum(e, axis=1, keepdims=True)

    return pl.pallas_call(
        body,
        grid=(n // _BLK,),
        in_specs=[pl.BlockSpec((2, _BLK, k), lambda i: (0, i, 0)),
                  pl.BlockSpec((1, ncls), lambda i: (0, 0))],
        out_specs=pl.BlockSpec((_BLK, ncls), lambda i: (i, 0)),
        out_shape=jax.ShapeDtypeStruct((n_out, ncls), jnp.float32),
    )(parts, b)


def kernel(x, edge_index, edge_weight, W1, b1, W2, b2):
    src = edge_index[0].astype(jnp.int32)
    dst = edge_index[1].astype(jnp.int32)
    ew = edge_weight.astype(jnp.float32)

    e = src.shape[0]
    tile = NUM_WORKERS * CHUNK * UNROLL
    e_pad = ((e + tile - 1) // tile) * tile
    pad = e_pad - e
    spread = jnp.arange(pad, dtype=jnp.int32) % jnp.int32(N_NODES)
    src_p = jnp.concatenate([src, spread])
    dst_p = jnp.concatenate([dst, spread])
    ew_p = jnp.pad(ew, (0, pad))
    n_chunks_w = e_pad // (NUM_WORKERS * CHUNK)

    nhid = W1.shape[1]
    ncls = W2.shape[1]
    zeros_h = jnp.zeros((ROWS_PER_SUB, nhid), jnp.float32)
    spmm = _make_spmm(nhid, n_chunks_w)
    w2_pad = jnp.pad(W2, ((0, 0), (0, nhid - ncls)))

    support = _tc_matmul(x, W1)
    parts1 = spmm(support, src_p, dst_p, ew_p, zeros_h)
    support2 = _tc_add_relu_matmul(parts1, b1.reshape(1, -1), w2_pad)
    parts2 = spmm(support2, src_p, dst_p, ew_p, zeros_h)
    return _tc_bias_softmax(parts2, b2.reshape(1, -1), N_NODES, ncls)


# speedup vs baseline: 9.9216x; 1.1934x over previous
import functools

import jax
import jax.numpy as jnp
from jax import lax
from jax.experimental import pallas as pl
from jax.experimental.pallas import tpu as pltpu
from jax.experimental.pallas import tpu_sc as plsc

N_NODES = 10000
NUM_CORES = 2
NUM_SUBCORES = 16
NUM_WORKERS = NUM_CORES * NUM_SUBCORES
CHUNK = 112
N_PAD = 10240
ROWS_PER_SUB = N_PAD // NUM_SUBCORES


NBUF = 3
NSRC = 3
NDST = 4
NWGT = 3
UNROLL = 12


@functools.lru_cache(maxsize=None)
def _make_spmm(feat: int, n_chunks_w: int):
    assert n_chunks_w % UNROLL == 0
    n_outer = n_chunks_w // UNROLL
    mesh = plsc.VectorSubcoreMesh(core_axis_name="c", subcore_axis_name="s")

    @functools.partial(
        pl.kernel,
        out_type=jax.ShapeDtypeStruct((NUM_CORES, N_PAD, feat), jnp.float32),
        mesh=mesh,
        scratch_types=[
            pltpu.VMEM((NSRC, CHUNK), jnp.int32),
            pltpu.VMEM((NDST, CHUNK), jnp.int32),
            pltpu.VMEM((NWGT, CHUNK), jnp.float32),
            pltpu.VMEM((NBUF, CHUNK, feat), jnp.float32),
            pltpu.VMEM_SHARED((N_PAD, feat), jnp.float32),
        ] + [pltpu.SemaphoreType.DMA] * (NBUF + NBUF + NSRC + NDST + NWGT),
    )
    def spmm(sup_hbm, src_hbm, dst_hbm, w_hbm, zeros_hbm, out_hbm,
             src_v, dst_v, w_v, rows_v, acc, *sems):
        gsem = sems[:NBUF]
        ssem = sems[NBUF:2 * NBUF]
        isrc = sems[2 * NBUF:2 * NBUF + NSRC]
        idst = sems[2 * NBUF + NSRC:2 * NBUF + NSRC + NDST]
        iwgt = sems[2 * NBUF + NSRC + NDST:]
        c = lax.axis_index("c")
        s = lax.axis_index("s")
        wid = c * NUM_SUBCORES + s
        base_e = wid * n_chunks_w * CHUNK

        def stage(i, ps, pd, pw):
            off = base_e + i * CHUNK
            pltpu.async_copy(src_hbm.at[pl.ds(off, CHUNK)], src_v.at[ps],
                             isrc[ps])
            pltpu.async_copy(dst_hbm.at[pl.ds(off, CHUNK)], dst_v.at[pd],
                             idst[pd])
            pltpu.async_copy(w_hbm.at[pl.ds(off, CHUNK)], w_v.at[pw],
                             iwgt[pw])

        def src_wait(ps):
            pltpu.make_async_copy(src_hbm.at[pl.ds(0, CHUNK)], src_v.at[ps],
                                  isrc[ps]).wait()

        def dst_wait(pd):
            pltpu.make_async_copy(dst_hbm.at[pl.ds(0, CHUNK)], dst_v.at[pd],
                                  idst[pd]).wait()

        def wgt_wait(pw):
            pltpu.make_async_copy(w_hbm.at[pl.ds(0, CHUNK)], w_v.at[pw],
                                  iwgt[pw]).wait()

        def gather_start(b, ps):
            pltpu.async_copy(sup_hbm.at[src_v.at[ps]], rows_v.at[b], gsem[b])

        def gather_wait(b, ps):
            pltpu.make_async_copy(sup_hbm.at[src_v.at[ps]], rows_v.at[b],
                                  gsem[b]).wait()

        def scatter_start(b, pd):
            pltpu.async_copy(rows_v.at[b], acc.at[dst_v.at[pd]], ssem[b],
                             add=True)

        def scatter_wait(b, pd):
            pltpu.make_async_copy(rows_v.at[b], acc.at[dst_v.at[pd]],
                                  ssem[b]).wait()

        def scale(b, pw):
            @functools.partial(plsc.parallel_loop, 0, CHUNK // 16, unroll=2)
            def _(g):
                w16 = w_v[pw, pl.ds(g * 16, 16)]
                for j in range(16):
                    we = w16[j]
                    e = g * 16 + j
                    for f in range(feat // 16):
                        sl = pl.ds(f * 16, 16)
                        rows_v[b, e, sl] = rows_v[b, e, sl] * we

        pltpu.sync_copy(zeros_hbm, acc.at[pl.ds(s * ROWS_PER_SUB, ROWS_PER_SUB)])
        plsc.subcore_barrier()

        stage(0, 0, 0, 0)
        stage(1, 1, 1, 1)
        src_wait(0)
        gather_start(0, 0)

        def outer(t, carry):
            for p in range(UNROLL):
                i = t * UNROLL + p
                b = p % NBUF
                bn = (p + 1) % NBUF
                if p >= 2:
                    scatter_wait(bn, (p - 2) % NDST)
                else:
                    @pl.when(t >= 1)
                    def _():
                        scatter_wait(bn, (p - 2) % NDST)
                if p < UNROLL - 2:
                    stage(i + 2, (p + 2) % NSRC, (p + 2) % NDST,
                          (p + 2) % NWGT)
                else:
                    @pl.when(t < n_outer - 1)
                    def _():
                        stage(i + 2, (p + 2) % NSRC, (p + 2) % NDST,
                              (p + 2) % NWGT)
                if p < UNROLL - 1:
                    src_wait((p + 1) % NSRC)
                    gather_start(bn, (p + 1) % NSRC)
                else:
                    @pl.when(t < n_outer - 1)
                    def _():
                        src_wait((p + 1) % NSRC)
                        gather_start(bn, (p + 1) % NSRC)
                gather_wait(b, p % NSRC)
                wgt_wait(p % NWGT)
                scale(b, p % NWGT)
                dst_wait(p % NDST)
                scatter_start(b, p % NDST)
            return carry

        lax.fori_loop(0, n_outer, outer, 0)

        scatter_wait((n_chunks_w - 2) % NBUF, (n_chunks_w - 2) % NDST)
        scatter_wait((n_chunks_w - 1) % NBUF, (n_chunks_w - 1) % NDST)

        plsc.subcore_barrier()

        pltpu.sync_copy(acc.at[pl.ds(s * ROWS_PER_SUB, ROWS_PER_SUB)],
                        out_hbm.at[c, pl.ds(s * ROWS_PER_SUB, ROWS_PER_SUB)])

    return spmm


def _blk(n):
    return 1024 if n % 1024 == 0 else 1000


def _tc_matmul(x, w):
    n, k = x.shape
    m = w.shape[1]
    _BLK = _blk(n)

    def body(x_ref, w_ref, o_ref):
        o_ref[...] = jnp.dot(x_ref[...], w_ref[...],
                             preferred_element_type=jnp.float32)

    return pl.pallas_call(
        body,
        grid=(n // _BLK,),
        in_specs=[pl.BlockSpec((_BLK, k), lambda i: (i, 0)),
                  pl.BlockSpec((k, m), lambda i: (0, 0))],
        out_specs=pl.BlockSpec((_BLK, m), lambda i: (i, 0)),
        out_shape=jax.ShapeDtypeStruct((n, m), jnp.float32),
    )(x, w)


def _tc_add_relu_matmul(parts, b, w):
    _, n, k = parts.shape
    m = w.shape[1]
    _BLK = _blk(n)

    def body(p_ref, b_ref, w_ref, o_ref):
        p = p_ref[...]
        h = jnp.maximum(p[0] + p[1] + b_ref[...], 0.0)
        o_ref[...] = jnp.dot(h, w_ref[...], preferred_element_type=jnp.float32)

    return pl.pallas_call(
        body,
        grid=(n // _BLK,),
        in_specs=[pl.BlockSpec((2, _BLK, k), lambda i: (0, i, 0)),
                  pl.BlockSpec((1, k), lambda i: (0, 0)),
                  pl.BlockSpec((k, m), lambda i: (0, 0))],
        out_specs=pl.BlockSpec((_BLK, m), lambda i: (i, 0)),
        out_shape=jax.ShapeDtypeStruct((n, m), jnp.float32),
    )(parts, b, w)


def _tc_bias_softmax(parts, b, n_out, ncls):
    _, n, k = parts.shape
    _BLK = _blk(n)

    def body(p_ref, b_ref, o_ref):
        p = p_ref[...]
        z = p[0, :, :ncls] + p[1, :, :ncls] + b_ref[...]
        z = z - jnp.max(z, axis=1, keepdims=True)
        e = jnp.exp(z)
        o_ref[...] = e / jnp.sum(e, axis=1, keepdims=True)

    return pl.pallas_call(
        body,
        grid=(n // _BLK,),
        in_specs=[pl.BlockSpec((2, _BLK, k), lambda i: (0, i, 0)),
                  pl.BlockSpec((1, ncls), lambda i: (0, 0))],
        out_specs=pl.BlockSpec((_BLK, ncls), lambda i: (i, 0)),
        out_shape=jax.ShapeDtypeStruct((n_out, ncls), jnp.float32),
    )(parts, b)


def kernel(x, edge_index, edge_weight, W1, b1, W2, b2):
    src = edge_index[0].astype(jnp.int32)
    dst = edge_index[1].astype(jnp.int32)
    ew = edge_weight.astype(jnp.float32)

    e = src.shape[0]
    tile = NUM_WORKERS * CHUNK * UNROLL
    e_pad = ((e + tile - 1) // tile) * tile
    pad = e_pad - e
    spread = jnp.arange(pad, dtype=jnp.int32) % jnp.int32(N_NODES)
    src_p = jnp.concatenate([src, spread])
    dst_p = jnp.concatenate([dst, spread])
    ew_p = jnp.pad(ew, (0, pad))
    n_chunks_w = e_pad // (NUM_WORKERS * CHUNK)

    nhid = W1.shape[1]
    ncls = W2.shape[1]
    zeros_h = jnp.zeros((ROWS_PER_SUB, nhid), jnp.float32)
    spmm = _make_spmm(nhid, n_chunks_w)
    w2_pad = jnp.pad(W2, ((0, 0), (0, nhid - ncls)))

    support = _tc_matmul(x, W1)
    parts1 = spmm(support, src_p, dst_p, ew_p, zeros_h)
    support2 = _tc_add_relu_matmul(parts1, b1.reshape(1, -1), w2_pad)
    parts2 = spmm(support2, src_p, dst_p, ew_p, zeros_h)
    return _tc_bias_softmax(parts2, b2.reshape(1, -1), N_NODES, ncls)
